# baseline (speedup 1.0000x reference)
import functools

import jax
import jax.numpy as jnp
from jax import lax
from jax.experimental import pallas as pl
from jax.experimental.pallas import tpu as pltpu
from jax.experimental.pallas import tpu_sc as plsc

N = 10000
E = 320000
D = 128
EMB = 132
EMBP = 144
NP = 10240
NAGG = 10112
RPA = NAGG // 16
NW = 32
EPW = 10240
EP = NW * EPW
EPP = EP + 2048

C = 64
NCH = EPW // C
SCH = 8
NSC = NCH // SCH

C1 = 128
SC1 = 2048
NSC1 = EPW // SC1

BLK = 256
NBLK = NP // BLK
EBLK = 2048
HI = lax.Precision.HIGHEST

_mesh = plsc.VectorSubcoreMesh(core_axis_name="c", subcore_axis_name="s")
_params = pltpu.CompilerParams(use_tc_tiling_on_sc=False)


@functools.partial(
    pl.kernel,
    out_type=jax.ShapeDtypeStruct((4 * NP, 16), jnp.float32),
    mesh=_mesh,
    scratch_types=[
        pltpu.VMEM((16, C1), jnp.int32),
        pltpu.VMEM((16, C1), jnp.int32),
        pltpu.VMEM((16, C1), jnp.int32),
        pltpu.VMEM((16, C1), jnp.int32),
        pltpu.VMEM((SC1, 16), jnp.float32),
        pltpu.VMEM((SC1, 16), jnp.float32),
        pltpu.VMEM_SHARED((NAGG, 16), jnp.float32),
        pltpu.VMEM_SHARED((NAGG, 16), jnp.float32),
        pltpu.SemaphoreType.DMA,
        pltpu.SemaphoreType.DMA,
        pltpu.SemaphoreType.DMA,
        pltpu.SemaphoreType.DMA,
    ],
    compiler_params=_params,
)
def _sc_stats(src2_h, dst2_h, e16_h, out_h,
              sx0, sx1, dx0, dx1, eb0, eb1, s_sh, d_sh,
              lsem0, lsem1, asem, bsem):
    c = lax.axis_index("c")
    s = lax.axis_index("s")
    wid = c * 16 + s
    sx = [sx0, sx1]
    dx = [dx0, dx1]
    eb = [eb0, eb1]
    lsem = [lsem0, lsem1]

    zv = jnp.zeros((16,), jnp.float32)

    def zb(i, carry):
        eb0[i, :] = zv
        return carry

    lax.fori_loop(0, RPA, zb, 0)
    r0 = s * RPA
    pltpu.sync_copy(eb0.at[pl.ds(0, RPA)], s_sh.at[pl.ds(r0, RPA)])
    pltpu.sync_copy(eb0.at[pl.ds(0, RPA)], d_sh.at[pl.ds(r0, RPA)])

    rb = wid * (EPW // C1)
    ebase = wid * EPW
    descs = []
    for q in (0, 1):
        descs.append([
            pltpu.async_copy(src2_h.at[pl.ds(rb + q * 16, 16)], sx[q],
                             lsem[q]),
            pltpu.async_copy(dst2_h.at[pl.ds(rb + q * 16, 16)], dx[q],
                             lsem[q]),
            pltpu.async_copy(e16_h.at[pl.ds(ebase + q * SC1, SC1)], eb[q],
                             lsem[q]),
        ])
    plsc.subcore_barrier()

    for sc in range(NSC1):
        q = sc % 2
        for dsc in descs[q]:
            dsc.wait()
        adds = []
        for j in range(16):
            adds.append(pltpu.async_copy(
                eb[q].at[pl.ds(j * C1, C1)], s_sh.at[sx[q].at[j]], asem,
                add=True))
            adds.append(pltpu.async_copy(
                eb[q].at[pl.ds(j * C1, C1)], d_sh.at[dx[q].at[j]], bsem,
                add=True))
        for a in adds:
            a.wait()
        if sc + 2 < NSC1:
            descs[q] = [
                pltpu.async_copy(src2_h.at[pl.ds(rb + (sc + 2) * 16, 16)],
                                 sx[q], lsem[q]),
                pltpu.async_copy(dst2_h.at[pl.ds(rb + (sc + 2) * 16, 16)],
                                 dx[q], lsem[q]),
                pltpu.async_copy(e16_h.at[pl.ds(ebase + (sc + 2) * SC1, SC1)],
                                 eb[q], lsem[q]),
            ]

    plsc.subcore_barrier()
    pltpu.sync_copy(s_sh.at[pl.ds(r0, RPA)], eb0.at[pl.ds(0, RPA)])
    pltpu.sync_copy(eb0.at[pl.ds(0, RPA)], out_h.at[pl.ds(c * NP + r0, RPA)])
    pltpu.sync_copy(d_sh.at[pl.ds(r0, RPA)], eb1.at[pl.ds(0, RPA)])
    pltpu.sync_copy(eb1.at[pl.ds(0, RPA)],
                    out_h.at[pl.ds(2 * NP + c * NP + r0, RPA)])

    @pl.when(s == 15)
    def _():
        def zt(i, carry):
            eb0[i, :] = zv
            return carry

        lax.fori_loop(0, NP - NAGG, zt, 0)
        tail = eb0.at[pl.ds(0, NP - NAGG)]
        pltpu.sync_copy(tail, out_h.at[pl.ds(c * NP + NAGG, NP - NAGG)])
        pltpu.sync_copy(tail,
                        out_h.at[pl.ds(2 * NP + c * NP + NAGG, NP - NAGG)])


@functools.partial(
    pl.kernel,
    out_type=jax.ShapeDtypeStruct((2 * NP, EMBP), jnp.float32),
    mesh=_mesh,
    scratch_types=[
        pltpu.VMEM((C,), jnp.int32),
        pltpu.VMEM((C,), jnp.int32),
        pltpu.VMEM((SCH, C), jnp.int32),
        pltpu.VMEM((SCH, C), jnp.int32),
        pltpu.VMEM((4 * SCH * C + 16,), jnp.float32),
        pltpu.VMEM((4 * SCH * C + 16,), jnp.float32),
        pltpu.VMEM((C, EMBP), jnp.float32),
        pltpu.VMEM((C, EMBP), jnp.float32),
        pltpu.VMEM((C, EMBP), jnp.float32),
        pltpu.VMEM((4, EMBP), jnp.float32),
        pltpu.VMEM_SHARED((NAGG, EMBP), jnp.float32),
        pltpu.SemaphoreType.DMA,
        pltpu.SemaphoreType.DMA,
        pltpu.SemaphoreType.DMA,
        pltpu.SemaphoreType.DMA,
        pltpu.SemaphoreType.DMA,
        pltpu.SemaphoreType.DMA,
    ],
    compiler_params=_params,
)
def _sc_edges(u_h, src_h, dst2_h, e_h, w1e_h, out_h,
              sb0, sb1, dxa, dxb, eba, ebb, ga, gb, stage, wv, agg_sh,
              gsem0, gsem1, ssem0, ssem1, lsem0, lsem1):
    c = lax.axis_index("c")
    s = lax.axis_index("s")
    wid = c * 16 + s
    sb = [sb0, sb1]
    dxq = [dxa, dxb]
    ebq = [eba, ebb]
    gq = [ga, gb]
    gsem = [gsem0, gsem1]
    ssem = [ssem0, ssem1]
    lsem = [lsem0, lsem1]

    pltpu.sync_copy(w1e_h, wv)

    zv = jnp.zeros((16,), jnp.float32)

    def zb(i, carry):
        for j in range(EMBP // 16):
            stage[i, pl.ds(16 * j, 16)] = zv
        return carry

    lax.fori_loop(0, C, zb, 0)
    r0 = s * RPA
    for t in range(RPA // C):
        pltpu.sync_copy(stage, agg_sh.at[pl.ds(r0 + t * C, C)])
    pltpu.sync_copy(stage.at[pl.ds(0, RPA - (RPA // C) * C)],
                    agg_sh.at[pl.ds(r0 + (RPA // C) * C,
                                    RPA - (RPA // C) * C)])
    plsc.subcore_barrier()

    wvec = [[wv[k, pl.ds(16 * j, 16)] for j in range(EMBP // 16)]
            for k in range(4)]

    ebase = wid * EPW
    drow = wid * NCH

    pltpu.sync_copy(dst2_h.at[pl.ds(drow, SCH)], dxq[0])
    pltpu.sync_copy(e_h.at[pl.ds(4 * ebase, 4 * SCH * C)],
                    eba.at[pl.ds(0, 4 * SCH * C)])
    pltpu.sync_copy(src_h.at[pl.ds(ebase, C)], sb0)
    pltpu.async_copy(u_h.at[sb0], ga, gsem0)
    pltpu.async_copy(src_h.at[pl.ds(ebase + C, C)], sb1, ssem1)
    pltpu.async_copy(dst2_h.at[pl.ds(drow + SCH, SCH)], dxb, lsem1)
    pltpu.async_copy(e_h.at[pl.ds(4 * (ebase + SCH * C), 4 * SCH * C)],
                     ebb.at[pl.ds(0, 4 * SCH * C)], lsem1)

    def wait_gather(p):
        pltpu.make_async_copy(u_h.at[sb[p]], gq[p], gsem[p]).wait()

    def wait_sidx(p):
        pltpu.make_async_copy(src_h.at[pl.ds(0, C)], sb[p], ssem[p]).wait()

    def wait_super(q):
        pltpu.make_async_copy(dst2_h.at[pl.ds(0, SCH)], dxq[q],
                              lsem[q]).wait()
        pltpu.make_async_copy(e_h.at[pl.ds(0, 4 * SCH * C)],
                              ebq[q].at[pl.ds(0, 4 * SCH * C)],
                              lsem[q]).wait()

    def pair(i, carry):
        for sc_off in (0, 1):
            q = sc_off
            sc = 2 * i + sc_off
            for j in range(SCH):
                p = j % 2
                t = sc * SCH + j
                wait_gather(p)
                wait_sidx(1 - p)
                if j == SCH - 1:
                    wait_super(1 - q)
                pltpu.async_copy(u_h.at[sb[1 - p]], gq[1 - p], gsem[1 - p])
                pltpu.async_copy(
                    src_h.at[pl.ds(ebase + (t + 2) * C, C)], sb[p], ssem[p])

                gath = gq[p]
                ech = ebq[q]

                def edge(ii, icarry):
                    k = j * C + ii
                    ev = ech[pl.ds(4 * k, 16)]
                    e0 = ev[0]
                    e1 = ev[1]
                    e2 = ev[2]
                    e3 = ev[3]
                    for jj in range(EMBP // 16):
                        z = (gath[ii, pl.ds(16 * jj, 16)]
                             + e0 * wvec[0][jj] + e1 * wvec[1][jj]
                             + e2 * wvec[2][jj] + e3 * wvec[3][jj])
                        stage[ii, pl.ds(16 * jj, 16)] = jnp.maximum(z, 0.0)
                    return icarry

                lax.fori_loop(0, C, edge, 0, unroll=2)
                pltpu.sync_copy(stage, agg_sh.at[dxq[q].at[j]], add=True)
                if j == SCH - 1:
                    pltpu.async_copy(
                        dst2_h.at[pl.ds(drow + (sc + 2) * SCH, SCH)],
                        dxq[q], lsem[q])
                    pltpu.async_copy(
                        e_h.at[pl.ds(4 * (ebase + (sc + 2) * SCH * C),
                                     4 * SCH * C)],
                        ebq[q].at[pl.ds(0, 4 * SCH * C)], lsem[q])
        return carry

    lax.fori_loop(0, NSC // 2, pair, 0)

    wait_gather(0)
    wait_sidx(1)
    wait_super(1)

    plsc.subcore_barrier()
    for t in range(RPA // C):
        pltpu.sync_copy(agg_sh.at[pl.ds(r0 + t * C, C)], stage)
        pltpu.sync_copy(stage, out_h.at[pl.ds(c * NP + r0 + t * C, C)])
    rem = RPA - (RPA // C) * C
    pltpu.sync_copy(agg_sh.at[pl.ds(r0 + (RPA // C) * C, rem)],
                    stage.at[pl.ds(0, rem)])
    pltpu.sync_copy(stage.at[pl.ds(0, rem)],
                    out_h.at[pl.ds(c * NP + r0 + (RPA // C) * C, rem)])


def _ka_body(x_ref, s0_ref, s1_ref, cxx_ref, mxe_ref):
    @pl.when(pl.program_id(0) == 0)
    def _():
        cxx_ref[...] = jnp.zeros_like(cxx_ref)
        mxe_ref[...] = jnp.zeros_like(mxe_ref)

    sblk = s0_ref[...] + s1_ref[...]
    xb = x_ref[...]
    w = sblk[:, 4:5]
    cxx_ref[...] += lax.dot_general(xb, xb * w, (((0,), (0,)), ((), ())),
                                    precision=HI,
                                    preferred_element_type=jnp.float32)
    mxe_ref[...] += lax.dot_general(xb, sblk, (((0,), (0,)), ((), ())),
                                    precision=HI,
                                    preferred_element_type=jnp.float32)


def _kb_body(e16_ref, g_ref):
    @pl.when(pl.program_id(0) == 0)
    def _():
        g_ref[...] = jnp.zeros_like(g_ref)

    eb = e16_ref[...]
    g_ref[...] += lax.dot_general(eb, eb, (((0,), (0,)), ((), ())),
                                  precision=HI,
                                  preferred_element_type=jnp.float32)


def _kf_body(cxx_ref, mxe_ref, g_ref, w1_ref, gam_ref, bet_ref,
             w1sp_ref, cpp_ref):
    cxe = mxe_ref[:, 0:4]
    sum_x = mxe_ref[:, 4:5]
    sum_e = g_ref[0:4, 4:5]
    w1 = w1_ref[...]
    w1x = w1[0:128, :]
    w1e = w1[128:132, :]
    dn = (((0,), (0,)), ((), ()))
    t = (lax.dot_general(sum_x, w1x, dn, precision=HI)
         + lax.dot_general(sum_e, w1e, dn, precision=HI))
    top = (lax.dot_general(cxx_ref[...], w1x, (((1,), (0,)), ((), ())),
                           precision=HI)
           + lax.dot_general(cxe, w1e, (((1,), (0,)), ((), ())),
                             precision=HI))
    bot = (lax.dot_general(cxe, w1x, dn, precision=HI)
           + lax.dot_general(g_ref[0:4, 0:4], w1e,
                             (((1,), (0,)), ((), ())), precision=HI))
    ss = (jnp.sum(w1x * top, axis=0, keepdims=True)
          + jnp.sum(w1e * bot, axis=0, keepdims=True))
    tm = t * (1.0 / E)
    var = ss * (1.0 / E) - tm * tm
    sfac = gam_ref[...] * lax.rsqrt(var + 1e-5)
    cp = bet_ref[...] - tm * sfac
    w1s = w1 * sfac
    pad_c = jnp.zeros((132, EMBP - EMB), jnp.float32)
    w1sp_ref[...] = jnp.concatenate([w1s, pad_c], axis=1)
    cp_row = jnp.concatenate([cp, jnp.zeros((1, EMBP - EMB), jnp.float32)],
                             axis=1)
    cpp_ref[...] = jnp.concatenate([cp_row, jnp.zeros((7, EMBP), jnp.float32)],
                                   axis=0)


def _ku_body(x_ref, w1sp_ref, cpp_ref, u_ref):
    u_ref[...] = lax.dot_general(x_ref[...], w1sp_ref[0:128, :],
                                 (((1,), (0,)), ((), ())), precision=HI,
                                 preferred_element_type=jnp.float32) \
                 + cpp_ref[0:1, :]


def _ko_body(a0_ref, a1_ref, d0_ref, d1_ref, x_ref, w2p_ref, b2_ref, h_ref):
    aggb = a0_ref[...] + a1_ref[...]
    deg = d0_ref[:, 4:5] + d1_ref[:, 4:5]
    h = lax.dot_general(aggb, w2p_ref[...], (((1,), (0,)), ((), ())),
                        precision=HI, preferred_element_type=jnp.float32)
    h = h + deg * b2_ref[...]
    h_ref[...] = jnp.where(deg > 0.0, h, x_ref[...])


def kernel(x, edge_index, e, W1, b1, gamma, beta, W2, b2):
    del b1
    src = edge_index[0]
    dst = edge_index[1]
    pad = EPP - E
    sent = jnp.full((pad,), N, jnp.int32)
    src_p = jnp.concatenate([src, sent])
    dst_p = jnp.concatenate([dst, sent])
    e_p = jnp.concatenate(
        [e, jnp.zeros((pad, 4), jnp.float32)]).reshape(-1)
    e16 = jnp.concatenate(
        [e, jnp.ones((E, 1), jnp.float32), jnp.zeros((E, 11), jnp.float32)],
        axis=1)
    e16 = jnp.concatenate([e16, jnp.zeros((EP - E, 16), jnp.float32)])
    x_p = jnp.concatenate([x, jnp.zeros((NP - N, D), jnp.float32)])
    src2 = src_p.reshape(-1, C1)
    dst2c1 = dst_p.reshape(-1, C1)
    dst2 = dst_p.reshape(-1, C)

    sd = _sc_stats(src2, dst2c1, e16)

    cxx, mxe = pl.pallas_call(
        _ka_body,
        grid=(NBLK,),
        in_specs=[
            pl.BlockSpec((BLK, D), lambda i: (i, 0)),
            pl.BlockSpec((BLK, 16), lambda i: (i, 0)),
            pl.BlockSpec((BLK, 16), lambda i: (i + NBLK, 0)),
        ],
        out_specs=[
            pl.BlockSpec((D, D), lambda i: (0, 0)),
            pl.BlockSpec((D, 16), lambda i: (0, 0)),
        ],
        out_shape=[
            jax.ShapeDtypeStruct((D, D), jnp.float32),
            jax.ShapeDtypeStruct((D, 16), jnp.float32),
        ],
    )(x_p, sd, sd)

    g = pl.pallas_call(
        _kb_body,
        grid=(EP // EBLK,),
        in_specs=[pl.BlockSpec((EBLK, 16), lambda i: (i, 0))],
        out_specs=pl.BlockSpec((16, 16), lambda i: (0, 0)),
        out_shape=jax.ShapeDtypeStruct((16, 16), jnp.float32),
    )(e16)

    w1sp, cpp = pl.pallas_call(
        _kf_body,
        out_shape=[
            jax.ShapeDtypeStruct((EMB, EMBP), jnp.float32),
            jax.ShapeDtypeStruct((8, EMBP), jnp.float32),
        ],
    )(cxx, mxe, g, W1, gamma.reshape(1, EMB), beta.reshape(1, EMB))

    u = pl.pallas_call(
        _ku_body,
        grid=(NBLK,),
        in_specs=[
            pl.BlockSpec((BLK, D), lambda i: (i, 0)),
            pl.BlockSpec((EMB, EMBP), lambda i: (0, 0)),
            pl.BlockSpec((8, EMBP), lambda i: (0, 0)),
        ],
        out_specs=pl.BlockSpec((BLK, EMBP), lambda i: (i, 0)),
        out_shape=jax.ShapeDtypeStruct((NP, EMBP), jnp.float32),
    )(x_p, w1sp, cpp)

    w1e_s = w1sp[128:132, :]
    agg = _sc_edges(u, src_p, dst2, e_p, w1e_s)

    w2p = jnp.concatenate([W2, jnp.zeros((EMBP - EMB, D), jnp.float32)])
    h = pl.pallas_call(
        _ko_body,
        grid=(NBLK,),
        in_specs=[
            pl.BlockSpec((BLK, EMBP), lambda i: (i, 0)),
            pl.BlockSpec((BLK, EMBP), lambda i: (i + NBLK, 0)),
            pl.BlockSpec((BLK, 16), lambda i: (i + 2 * NBLK, 0)),
            pl.BlockSpec((BLK, 16), lambda i: (i + 3 * NBLK, 0)),
            pl.BlockSpec((BLK, D), lambda i: (i, 0)),
            pl.BlockSpec((EMBP, D), lambda i: (0, 0)),
            pl.BlockSpec((1, D), lambda i: (0, 0)),
        ],
        out_specs=pl.BlockSpec((BLK, D), lambda i: (i, 0)),
        out_shape=jax.ShapeDtypeStruct((NP, D), jnp.float32),
    )(agg, agg, sd, sd, x_p, w2p, b2.reshape(1, D))

    return h[:N]


# speedup vs baseline: 1.0228x; 1.0228x over previous
import functools

import jax
import jax.numpy as jnp
from jax import lax
from jax.experimental import pallas as pl
from jax.experimental.pallas import tpu as pltpu
from jax.experimental.pallas import tpu_sc as plsc

N = 10000
E = 320000
D = 128
EMB = 132
EMBP = 144
NP = 10240
NAGG = 10112
RPA = NAGG // 16
NW = 32
EPW = 10240
EP = NW * EPW
EPP = EP + 2048

C = 64
NCH = EPW // C
SCH = 8
NSC = NCH // SCH

C1 = 128
SC1 = 2048
NSC1 = EPW // SC1

BLK = 256
NBLK = NP // BLK
EBLK = 2048
HI = lax.Precision.HIGHEST

_mesh = plsc.VectorSubcoreMesh(core_axis_name="c", subcore_axis_name="s")
_params = pltpu.CompilerParams(use_tc_tiling_on_sc=False)


@functools.partial(
    pl.kernel,
    out_type=jax.ShapeDtypeStruct((4 * NP, 16), jnp.float32),
    mesh=_mesh,
    scratch_types=[
        pltpu.VMEM((16, C1), jnp.int32),
        pltpu.VMEM((16, C1), jnp.int32),
        pltpu.VMEM((16, C1), jnp.int32),
        pltpu.VMEM((16, C1), jnp.int32),
        pltpu.VMEM((SC1, 16), jnp.float32),
        pltpu.VMEM((SC1, 16), jnp.float32),
        pltpu.VMEM_SHARED((NAGG, 16), jnp.float32),
        pltpu.VMEM_SHARED((NAGG, 16), jnp.float32),
        pltpu.SemaphoreType.DMA,
        pltpu.SemaphoreType.DMA,
        pltpu.SemaphoreType.DMA,
        pltpu.SemaphoreType.DMA,
    ],
    compiler_params=_params,
)
def _sc_stats(src2_h, dst2_h, e16_h, out_h,
              sx0, sx1, dx0, dx1, eb0, eb1, s_sh, d_sh,
              lsem0, lsem1, asem, bsem):
    c = lax.axis_index("c")
    s = lax.axis_index("s")
    wid = c * 16 + s
    sx = [sx0, sx1]
    dx = [dx0, dx1]
    eb = [eb0, eb1]
    lsem = [lsem0, lsem1]

    zv = jnp.zeros((16,), jnp.float32)

    def zb(i, carry):
        eb0[i, :] = zv
        return carry

    lax.fori_loop(0, RPA, zb, 0)
    r0 = s * RPA
    pltpu.sync_copy(eb0.at[pl.ds(0, RPA)], s_sh.at[pl.ds(r0, RPA)])
    pltpu.sync_copy(eb0.at[pl.ds(0, RPA)], d_sh.at[pl.ds(r0, RPA)])

    rb = wid * (EPW // C1)
    ebase = wid * EPW
    descs = []
    for q in (0, 1):
        descs.append([
            pltpu.async_copy(src2_h.at[pl.ds(rb + q * 16, 16)], sx[q],
                             lsem[q]),
            pltpu.async_copy(dst2_h.at[pl.ds(rb + q * 16, 16)], dx[q],
                             lsem[q]),
            pltpu.async_copy(e16_h.at[pl.ds(ebase + q * SC1, SC1)], eb[q],
                             lsem[q]),
        ])
    plsc.subcore_barrier()

    for sc in range(NSC1):
        q = sc % 2
        for dsc in descs[q]:
            dsc.wait()
        adds = []
        for j in range(16):
            adds.append(pltpu.async_copy(
                eb[q].at[pl.ds(j * C1, C1)], s_sh.at[sx[q].at[j]], asem,
                add=True))
            adds.append(pltpu.async_copy(
                eb[q].at[pl.ds(j * C1, C1)], d_sh.at[dx[q].at[j]], bsem,
                add=True))
        for a in adds:
            a.wait()
        if sc + 2 < NSC1:
            descs[q] = [
                pltpu.async_copy(src2_h.at[pl.ds(rb + (sc + 2) * 16, 16)],
                                 sx[q], lsem[q]),
                pltpu.async_copy(dst2_h.at[pl.ds(rb + (sc + 2) * 16, 16)],
                                 dx[q], lsem[q]),
                pltpu.async_copy(e16_h.at[pl.ds(ebase + (sc + 2) * SC1, SC1)],
                                 eb[q], lsem[q]),
            ]

    plsc.subcore_barrier()
    pltpu.sync_copy(s_sh.at[pl.ds(r0, RPA)], eb0.at[pl.ds(0, RPA)])
    pltpu.sync_copy(eb0.at[pl.ds(0, RPA)], out_h.at[pl.ds(c * NP + r0, RPA)])
    pltpu.sync_copy(d_sh.at[pl.ds(r0, RPA)], eb1.at[pl.ds(0, RPA)])
    pltpu.sync_copy(eb1.at[pl.ds(0, RPA)],
                    out_h.at[pl.ds(2 * NP + c * NP + r0, RPA)])

    @pl.when(s == 15)
    def _():
        def zt(i, carry):
            eb0[i, :] = zv
            return carry

        lax.fori_loop(0, NP - NAGG, zt, 0)
        tail = eb0.at[pl.ds(0, NP - NAGG)]
        pltpu.sync_copy(tail, out_h.at[pl.ds(c * NP + NAGG, NP - NAGG)])
        pltpu.sync_copy(tail,
                        out_h.at[pl.ds(2 * NP + c * NP + NAGG, NP - NAGG)])


@functools.partial(
    pl.kernel,
    out_type=jax.ShapeDtypeStruct((2 * NP, EMBP), jnp.float32),
    mesh=_mesh,
    scratch_types=[
        pltpu.VMEM((C,), jnp.int32),
        pltpu.VMEM((C,), jnp.int32),
        pltpu.VMEM((SCH, C), jnp.int32),
        pltpu.VMEM((SCH, C), jnp.int32),
        pltpu.VMEM((4 * SCH * C + 16,), jnp.float32),
        pltpu.VMEM((4 * SCH * C + 16,), jnp.float32),
        pltpu.VMEM((C, EMBP), jnp.float32),
        pltpu.VMEM((C, EMBP), jnp.float32),
        pltpu.VMEM((C, EMBP), jnp.float32),
        pltpu.VMEM((4, EMBP), jnp.float32),
        pltpu.VMEM_SHARED((NAGG, EMBP), jnp.float32),
        pltpu.SemaphoreType.DMA,
        pltpu.SemaphoreType.DMA,
        pltpu.SemaphoreType.DMA,
        pltpu.SemaphoreType.DMA,
        pltpu.SemaphoreType.DMA,
        pltpu.SemaphoreType.DMA,
    ],
    compiler_params=_params,
)
def _sc_edges(u_h, src_h, dst2_h, e_h, w1e_h, out_h,
              sb0, sb1, dxa, dxb, eba, ebb, ga, gb, stage, wv, agg_sh,
              gsem0, gsem1, ssem0, ssem1, lsem0, lsem1):
    c = lax.axis_index("c")
    s = lax.axis_index("s")
    wid = c * 16 + s
    sb = [sb0, sb1]
    dxq = [dxa, dxb]
    ebq = [eba, ebb]
    gq = [ga, gb]
    gsem = [gsem0, gsem1]
    ssem = [ssem0, ssem1]
    lsem = [lsem0, lsem1]

    pltpu.sync_copy(w1e_h, wv)

    zv = jnp.zeros((16,), jnp.float32)

    def zb(i, carry):
        for j in range(EMBP // 16):
            stage[i, pl.ds(16 * j, 16)] = zv
        return carry

    lax.fori_loop(0, C, zb, 0)
    r0 = s * RPA
    for t in range(RPA // C):
        pltpu.sync_copy(stage, agg_sh.at[pl.ds(r0 + t * C, C)])
    pltpu.sync_copy(stage.at[pl.ds(0, RPA - (RPA // C) * C)],
                    agg_sh.at[pl.ds(r0 + (RPA // C) * C,
                                    RPA - (RPA // C) * C)])
    plsc.subcore_barrier()

    wvec = [[wv[k, pl.ds(16 * j, 16)] for j in range(EMBP // 16)]
            for k in range(4)]

    ebase = wid * EPW
    drow = wid * NCH

    pltpu.sync_copy(dst2_h.at[pl.ds(drow, SCH)], dxq[0])
    pltpu.sync_copy(e_h.at[pl.ds(4 * ebase, 4 * SCH * C)],
                    eba.at[pl.ds(0, 4 * SCH * C)])
    pltpu.sync_copy(src_h.at[pl.ds(ebase, C)], sb0)
    pltpu.async_copy(u_h.at[sb0], ga, gsem0)
    pltpu.async_copy(src_h.at[pl.ds(ebase + C, C)], sb1, ssem1)
    pltpu.async_copy(dst2_h.at[pl.ds(drow + SCH, SCH)], dxb, lsem1)
    pltpu.async_copy(e_h.at[pl.ds(4 * (ebase + SCH * C), 4 * SCH * C)],
                     ebb.at[pl.ds(0, 4 * SCH * C)], lsem1)

    def wait_gather(p):
        pltpu.make_async_copy(u_h.at[sb[p]], gq[p], gsem[p]).wait()

    def wait_sidx(p):
        pltpu.make_async_copy(src_h.at[pl.ds(0, C)], sb[p], ssem[p]).wait()

    def wait_super(q):
        pltpu.make_async_copy(dst2_h.at[pl.ds(0, SCH)], dxq[q],
                              lsem[q]).wait()
        pltpu.make_async_copy(e_h.at[pl.ds(0, 4 * SCH * C)],
                              ebq[q].at[pl.ds(0, 4 * SCH * C)],
                              lsem[q]).wait()

    def pair(i, carry):
        for sc_off in (0, 1):
            q = sc_off
            sc = 2 * i + sc_off
            for j in range(SCH):
                p = j % 2
                t = sc * SCH + j
                wait_gather(p)
                wait_sidx(1 - p)
                if j == SCH - 1:
                    wait_super(1 - q)
                pltpu.async_copy(u_h.at[sb[1 - p]], gq[1 - p], gsem[1 - p])
                pltpu.async_copy(
                    src_h.at[pl.ds(ebase + (t + 2) * C, C)], sb[p], ssem[p])

                gath = gq[p]
                ech = ebq[q]

                def edge(ii, icarry):
                    k = j * C + ii
                    ev = ech[pl.ds(4 * k, 16)]
                    e0 = ev[0]
                    e1 = ev[1]
                    e2 = ev[2]
                    e3 = ev[3]
                    for jj in range(EMBP // 16):
                        z = (gath[ii, pl.ds(16 * jj, 16)]
                             + e0 * wvec[0][jj] + e1 * wvec[1][jj]
                             + e2 * wvec[2][jj] + e3 * wvec[3][jj])
                        stage[ii, pl.ds(16 * jj, 16)] = jnp.maximum(z, 0.0)
                    return icarry

                lax.fori_loop(0, C, edge, 0)
                pltpu.sync_copy(stage, agg_sh.at[dxq[q].at[j]], add=True)
                if j == SCH - 1:
                    pltpu.async_copy(
                        dst2_h.at[pl.ds(drow + (sc + 2) * SCH, SCH)],
                        dxq[q], lsem[q])
                    pltpu.async_copy(
                        e_h.at[pl.ds(4 * (ebase + (sc + 2) * SCH * C),
                                     4 * SCH * C)],
                        ebq[q].at[pl.ds(0, 4 * SCH * C)], lsem[q])
        return carry

    lax.fori_loop(0, NSC // 2, pair, 0)

    wait_gather(0)
    wait_sidx(1)
    wait_super(1)

    plsc.subcore_barrier()
    for t in range(RPA // C):
        pltpu.sync_copy(agg_sh.at[pl.ds(r0 + t * C, C)], stage)
        pltpu.sync_copy(stage, out_h.at[pl.ds(c * NP + r0 + t * C, C)])
    rem = RPA - (RPA // C) * C
    pltpu.sync_copy(agg_sh.at[pl.ds(r0 + (RPA // C) * C, rem)],
                    stage.at[pl.ds(0, rem)])
    pltpu.sync_copy(stage.at[pl.ds(0, rem)],
                    out_h.at[pl.ds(c * NP + r0 + (RPA // C) * C, rem)])


def _ka_body(x_ref, s0_ref, s1_ref, cxx_ref, mxe_ref):
    @pl.when(pl.program_id(0) == 0)
    def _():
        cxx_ref[...] = jnp.zeros_like(cxx_ref)
        mxe_ref[...] = jnp.zeros_like(mxe_ref)

    sblk = s0_ref[...] + s1_ref[...]
    xb = x_ref[...]
    w = sblk[:, 4:5]
    cxx_ref[...] += lax.dot_general(xb, xb * w, (((0,), (0,)), ((), ())),
                                    precision=HI,
                                    preferred_element_type=jnp.float32)
    mxe_ref[...] += lax.dot_general(xb, sblk, (((0,), (0,)), ((), ())),
                                    precision=HI,
                                    preferred_element_type=jnp.float32)


def _kb_body(e16_ref, g_ref):
    @pl.when(pl.program_id(0) == 0)
    def _():
        g_ref[...] = jnp.zeros_like(g_ref)

    eb = e16_ref[...]
    g_ref[...] += lax.dot_general(eb, eb, (((0,), (0,)), ((), ())),
                                  precision=HI,
                                  preferred_element_type=jnp.float32)


def _kf_body(cxx_ref, mxe_ref, g_ref, w1_ref, gam_ref, bet_ref,
             w1sp_ref, cpp_ref):
    cxe = mxe_ref[:, 0:4]
    sum_x = mxe_ref[:, 4:5]
    sum_e = g_ref[0:4, 4:5]
    w1 = w1_ref[...]
    w1x = w1[0:128, :]
    w1e = w1[128:132, :]
    dn = (((0,), (0,)), ((), ()))
    t = (lax.dot_general(sum_x, w1x, dn, precision=HI)
         + lax.dot_general(sum_e, w1e, dn, precision=HI))
    top = (lax.dot_general(cxx_ref[...], w1x, (((1,), (0,)), ((), ())),
                           precision=HI)
           + lax.dot_general(cxe, w1e, (((1,), (0,)), ((), ())),
                             precision=HI))
    bot = (lax.dot_general(cxe, w1x, dn, precision=HI)
           + lax.dot_general(g_ref[0:4, 0:4], w1e,
                             (((1,), (0,)), ((), ())), precision=HI))
    ss = (jnp.sum(w1x * top, axis=0, keepdims=True)
          + jnp.sum(w1e * bot, axis=0, keepdims=True))
    tm = t * (1.0 / E)
    var = ss * (1.0 / E) - tm * tm
    sfac = gam_ref[...] * lax.rsqrt(var + 1e-5)
    cp = bet_ref[...] - tm * sfac
    w1s = w1 * sfac
    pad_c = jnp.zeros((132, EMBP - EMB), jnp.float32)
    w1sp_ref[...] = jnp.concatenate([w1s, pad_c], axis=1)
    cp_row = jnp.concatenate([cp, jnp.zeros((1, EMBP - EMB), jnp.float32)],
                             axis=1)
    cpp_ref[...] = jnp.concatenate([cp_row, jnp.zeros((7, EMBP), jnp.float32)],
                                   axis=0)


def _ku_body(x_ref, w1sp_ref, cpp_ref, u_ref):
    u_ref[...] = lax.dot_general(x_ref[...], w1sp_ref[0:128, :],
                                 (((1,), (0,)), ((), ())), precision=HI,
                                 preferred_element_type=jnp.float32) \
                 + cpp_ref[0:1, :]


def _ko_body(a0_ref, a1_ref, d0_ref, d1_ref, x_ref, w2p_ref, b2_ref, h_ref):
    aggb = a0_ref[...] + a1_ref[...]
    deg = d0_ref[:, 4:5] + d1_ref[:, 4:5]
    h = lax.dot_general(aggb, w2p_ref[...], (((1,), (0,)), ((), ())),
                        precision=HI, preferred_element_type=jnp.float32)
    h = h + deg * b2_ref[...]
    h_ref[...] = jnp.where(deg > 0.0, h, x_ref[...])


def kernel(x, edge_index, e, W1, b1, gamma, beta, W2, b2):
    del b1
    src = edge_index[0]
    dst = edge_index[1]
    pad = EPP - E
    sent = jnp.full((pad,), N, jnp.int32)
    src_p = jnp.concatenate([src, sent])
    dst_p = jnp.concatenate([dst, sent])
    e_p = jnp.concatenate(
        [e, jnp.zeros((pad, 4), jnp.float32)]).reshape(-1)
    e16 = jnp.concatenate(
        [e, jnp.ones((E, 1), jnp.float32), jnp.zeros((E, 11), jnp.float32)],
        axis=1)
    e16 = jnp.concatenate([e16, jnp.zeros((EP - E, 16), jnp.float32)])
    x_p = jnp.concatenate([x, jnp.zeros((NP - N, D), jnp.float32)])
    src2 = src_p.reshape(-1, C1)
    dst2c1 = dst_p.reshape(-1, C1)
    dst2 = dst_p.reshape(-1, C)

    sd = _sc_stats(src2, dst2c1, e16)

    cxx, mxe = pl.pallas_call(
        _ka_body,
        grid=(NBLK,),
        in_specs=[
            pl.BlockSpec((BLK, D), lambda i: (i, 0)),
            pl.BlockSpec((BLK, 16), lambda i: (i, 0)),
            pl.BlockSpec((BLK, 16), lambda i: (i + NBLK, 0)),
        ],
        out_specs=[
            pl.BlockSpec((D, D), lambda i: (0, 0)),
            pl.BlockSpec((D, 16), lambda i: (0, 0)),
        ],
        out_shape=[
            jax.ShapeDtypeStruct((D, D), jnp.float32),
            jax.ShapeDtypeStruct((D, 16), jnp.float32),
        ],
    )(x_p, sd, sd)

    g = pl.pallas_call(
        _kb_body,
        grid=(EP // EBLK,),
        in_specs=[pl.BlockSpec((EBLK, 16), lambda i: (i, 0))],
        out_specs=pl.BlockSpec((16, 16), lambda i: (0, 0)),
        out_shape=jax.ShapeDtypeStruct((16, 16), jnp.float32),
    )(e16)

    w1sp, cpp = pl.pallas_call(
        _kf_body,
        out_shape=[
            jax.ShapeDtypeStruct((EMB, EMBP), jnp.float32),
            jax.ShapeDtypeStruct((8, EMBP), jnp.float32),
        ],
    )(cxx, mxe, g, W1, gamma.reshape(1, EMB), beta.reshape(1, EMB))

    u = pl.pallas_call(
        _ku_body,
        grid=(NBLK,),
        in_specs=[
            pl.BlockSpec((BLK, D), lambda i: (i, 0)),
            pl.BlockSpec((EMB, EMBP), lambda i: (0, 0)),
            pl.BlockSpec((8, EMBP), lambda i: (0, 0)),
        ],
        out_specs=pl.BlockSpec((BLK, EMBP), lambda i: (i, 0)),
        out_shape=jax.ShapeDtypeStruct((NP, EMBP), jnp.float32),
    )(x_p, w1sp, cpp)

    w1e_s = w1sp[128:132, :]
    agg = _sc_edges(u, src_p, dst2, e_p, w1e_s)

    w2p = jnp.concatenate([W2, jnp.zeros((EMBP - EMB, D), jnp.float32)])
    h = pl.pallas_call(
        _ko_body,
        grid=(NBLK,),
        in_specs=[
            pl.BlockSpec((BLK, EMBP), lambda i: (i, 0)),
            pl.BlockSpec((BLK, EMBP), lambda i: (i + NBLK, 0)),
            pl.BlockSpec((BLK, 16), lambda i: (i + 2 * NBLK, 0)),
            pl.BlockSpec((BLK, 16), lambda i: (i + 3 * NBLK, 0)),
            pl.BlockSpec((BLK, D), lambda i: (i, 0)),
            pl.BlockSpec((EMBP, D), lambda i: (0, 0)),
            pl.BlockSpec((1, D), lambda i: (0, 0)),
        ],
        out_specs=pl.BlockSpec((BLK, D), lambda i: (i, 0)),
        out_shape=jax.ShapeDtypeStruct((NP, D), jnp.float32),
    )(agg, agg, sd, sd, x_p, w2p, b2.reshape(1, D))

    return h[:N]


# speedup vs baseline: 1.1936x; 1.1670x over previous
import functools

import jax
import jax.numpy as jnp
from jax import lax
from jax.experimental import pallas as pl
from jax.experimental.pallas import tpu as pltpu
from jax.experimental.pallas import tpu_sc as plsc

N = 10000
E = 320000
D = 128
EMB = 132
EMBP = 144
NP = 10240
NAGG = 10112
RPA = NAGG // 16
NW = 32
EPW = 10240
EP = NW * EPW
EPP = EP + 2048

C = 64
NCH = EPW // C
SCH = 8
NSC = NCH // SCH

C1 = 128
SC1 = 2048
NSC1 = EPW // SC1

BLK = 256
NBLK = NP // BLK
EBLK = 2048
HI = lax.Precision.HIGHEST

_mesh = plsc.VectorSubcoreMesh(core_axis_name="c", subcore_axis_name="s")
_params = pltpu.CompilerParams(use_tc_tiling_on_sc=False)


@functools.partial(
    pl.kernel,
    out_type=jax.ShapeDtypeStruct((4 * NP, 16), jnp.float32),
    mesh=_mesh,
    scratch_types=[
        pltpu.VMEM((16, C1), jnp.int32),
        pltpu.VMEM((16, C1), jnp.int32),
        pltpu.VMEM((16, C1), jnp.int32),
        pltpu.VMEM((16, C1), jnp.int32),
        pltpu.VMEM((SC1, 16), jnp.float32),
        pltpu.VMEM((SC1, 16), jnp.float32),
        pltpu.VMEM_SHARED((NAGG, 16), jnp.float32),
        pltpu.VMEM_SHARED((NAGG, 16), jnp.float32),
        pltpu.SemaphoreType.DMA,
        pltpu.SemaphoreType.DMA,
        pltpu.SemaphoreType.DMA,
        pltpu.SemaphoreType.DMA,
    ],
    compiler_params=_params,
)
def _sc_stats(src2_h, dst2_h, e16_h, out_h,
              sx0, sx1, dx0, dx1, eb0, eb1, s_sh, d_sh,
              lsem0, lsem1, asem, bsem):
    c = lax.axis_index("c")
    s = lax.axis_index("s")
    wid = c * 16 + s
    sx = [sx0, sx1]
    dx = [dx0, dx1]
    eb = [eb0, eb1]
    lsem = [lsem0, lsem1]

    zv = jnp.zeros((16,), jnp.float32)

    def zb(i, carry):
        eb0[i, :] = zv
        return carry

    lax.fori_loop(0, RPA, zb, 0)
    r0 = s * RPA
    pltpu.sync_copy(eb0.at[pl.ds(0, RPA)], s_sh.at[pl.ds(r0, RPA)])
    pltpu.sync_copy(eb0.at[pl.ds(0, RPA)], d_sh.at[pl.ds(r0, RPA)])

    rb = wid * (EPW // C1)
    ebase = wid * EPW
    descs = []
    for q in (0, 1):
        descs.append([
            pltpu.async_copy(src2_h.at[pl.ds(rb + q * 16, 16)], sx[q],
                             lsem[q]),
            pltpu.async_copy(dst2_h.at[pl.ds(rb + q * 16, 16)], dx[q],
                             lsem[q]),
            pltpu.async_copy(e16_h.at[pl.ds(ebase + q * SC1, SC1)], eb[q],
                             lsem[q]),
        ])
    plsc.subcore_barrier()

    for sc in range(NSC1):
        q = sc % 2
        for dsc in descs[q]:
            dsc.wait()
        adds = []
        for j in range(16):
            adds.append(pltpu.async_copy(
                eb[q].at[pl.ds(j * C1, C1)], s_sh.at[sx[q].at[j]], asem,
                add=True))
            adds.append(pltpu.async_copy(
                eb[q].at[pl.ds(j * C1, C1)], d_sh.at[dx[q].at[j]], bsem,
                add=True))
        for a in adds:
            a.wait()
        if sc + 2 < NSC1:
            descs[q] = [
                pltpu.async_copy(src2_h.at[pl.ds(rb + (sc + 2) * 16, 16)],
                                 sx[q], lsem[q]),
                pltpu.async_copy(dst2_h.at[pl.ds(rb + (sc + 2) * 16, 16)],
                                 dx[q], lsem[q]),
                pltpu.async_copy(e16_h.at[pl.ds(ebase + (sc + 2) * SC1, SC1)],
                                 eb[q], lsem[q]),
            ]

    plsc.subcore_barrier()
    pltpu.sync_copy(s_sh.at[pl.ds(r0, RPA)], eb0.at[pl.ds(0, RPA)])
    pltpu.sync_copy(eb0.at[pl.ds(0, RPA)], out_h.at[pl.ds(c * NP + r0, RPA)])
    pltpu.sync_copy(d_sh.at[pl.ds(r0, RPA)], eb1.at[pl.ds(0, RPA)])
    pltpu.sync_copy(eb1.at[pl.ds(0, RPA)],
                    out_h.at[pl.ds(2 * NP + c * NP + r0, RPA)])

    @pl.when(s == 15)
    def _():
        def zt(i, carry):
            eb0[i, :] = zv
            return carry

        lax.fori_loop(0, NP - NAGG, zt, 0)
        tail = eb0.at[pl.ds(0, NP - NAGG)]
        pltpu.sync_copy(tail, out_h.at[pl.ds(c * NP + NAGG, NP - NAGG)])
        pltpu.sync_copy(tail,
                        out_h.at[pl.ds(2 * NP + c * NP + NAGG, NP - NAGG)])


@functools.partial(
    pl.kernel,
    out_type=jax.ShapeDtypeStruct((2 * NP, EMBP), jnp.float32),
    mesh=_mesh,
    scratch_types=[
        pltpu.VMEM((C,), jnp.int32),
        pltpu.VMEM((C,), jnp.int32),
        pltpu.VMEM((SCH, C), jnp.int32),
        pltpu.VMEM((SCH, C), jnp.int32),
        pltpu.VMEM((4 * SCH * C + 16,), jnp.float32),
        pltpu.VMEM((4 * SCH * C + 16,), jnp.float32),
        pltpu.VMEM((C, EMBP), jnp.float32),
        pltpu.VMEM((C, EMBP), jnp.float32),
        pltpu.VMEM((C, EMBP), jnp.float32),
        pltpu.VMEM((4, EMBP), jnp.float32),
        pltpu.VMEM_SHARED((NAGG, EMBP), jnp.float32),
        pltpu.SemaphoreType.DMA,
        pltpu.SemaphoreType.DMA,
        pltpu.SemaphoreType.DMA,
        pltpu.SemaphoreType.DMA,
        pltpu.SemaphoreType.DMA,
        pltpu.SemaphoreType.DMA,
    ],
    compiler_params=_params,
)
def _sc_edges(u_h, src_h, dst2_h, e_h, w1e_h, out_h,
              sb0, sb1, dxa, dxb, eba, ebb, ga, gb, stage, wv, agg_sh,
              gsem0, gsem1, ssem0, ssem1, lsem0, lsem1):
    c = lax.axis_index("c")
    s = lax.axis_index("s")
    wid = c * 16 + s
    sb = [sb0, sb1]
    dxq = [dxa, dxb]
    ebq = [eba, ebb]
    gq = [ga, gb]
    gsem = [gsem0, gsem1]
    ssem = [ssem0, ssem1]
    lsem = [lsem0, lsem1]

    pltpu.sync_copy(w1e_h, wv)

    zv = jnp.zeros((16,), jnp.float32)

    def zb(i, carry):
        for j in range(EMBP // 16):
            stage[i, pl.ds(16 * j, 16)] = zv
        return carry

    lax.fori_loop(0, C, zb, 0)
    r0 = s * RPA
    for t in range(RPA // C):
        pltpu.sync_copy(stage, agg_sh.at[pl.ds(r0 + t * C, C)])
    pltpu.sync_copy(stage.at[pl.ds(0, RPA - (RPA // C) * C)],
                    agg_sh.at[pl.ds(r0 + (RPA // C) * C,
                                    RPA - (RPA // C) * C)])
    plsc.subcore_barrier()

    wvec = [[wv[k, pl.ds(16 * j, 16)] for j in range(EMBP // 16)]
            for k in range(4)]

    ebase = wid * EPW
    drow = wid * NCH

    pltpu.sync_copy(dst2_h.at[pl.ds(drow, SCH)], dxq[0])
    pltpu.sync_copy(e_h.at[pl.ds(4 * ebase, 4 * SCH * C)],
                    eba.at[pl.ds(0, 4 * SCH * C)])
    pltpu.sync_copy(src_h.at[pl.ds(ebase, C)], sb0)
    pltpu.async_copy(u_h.at[sb0], ga, gsem0)
    pltpu.async_copy(src_h.at[pl.ds(ebase + C, C)], sb1, ssem1)
    pltpu.async_copy(dst2_h.at[pl.ds(drow + SCH, SCH)], dxb, lsem1)
    pltpu.async_copy(e_h.at[pl.ds(4 * (ebase + SCH * C), 4 * SCH * C)],
                     ebb.at[pl.ds(0, 4 * SCH * C)], lsem1)

    def wait_gather(p):
        pltpu.make_async_copy(u_h.at[sb[p]], gq[p], gsem[p]).wait()

    def wait_sidx(p):
        pltpu.make_async_copy(src_h.at[pl.ds(0, C)], sb[p], ssem[p]).wait()

    def wait_super(q):
        pltpu.make_async_copy(dst2_h.at[pl.ds(0, SCH)], dxq[q],
                              lsem[q]).wait()
        pltpu.make_async_copy(e_h.at[pl.ds(0, 4 * SCH * C)],
                              ebq[q].at[pl.ds(0, 4 * SCH * C)],
                              lsem[q]).wait()

    def pair(i, carry):
        for sc_off in (0, 1):
            q = sc_off
            sc = 2 * i + sc_off
            for j in range(SCH):
                p = j % 2
                t = sc * SCH + j
                wait_gather(p)
                wait_sidx(1 - p)
                if j == SCH - 1:
                    wait_super(1 - q)
                pltpu.async_copy(u_h.at[sb[1 - p]], gq[1 - p], gsem[1 - p])
                pltpu.async_copy(
                    src_h.at[pl.ds(ebase + (t + 2) * C, C)], sb[p], ssem[p])

                gath = gq[p]
                ech = ebq[q]

                @plsc.parallel_loop(0, C)
                def _(ii):
                    k = j * C + ii
                    ev = ech[pl.ds(4 * k, 16)]
                    e0 = ev[0]
                    e1 = ev[1]
                    e2 = ev[2]
                    e3 = ev[3]
                    for jj in range(EMBP // 16):
                        z = (gath[ii, pl.ds(16 * jj, 16)]
                             + e0 * wvec[0][jj] + e1 * wvec[1][jj]
                             + e2 * wvec[2][jj] + e3 * wvec[3][jj])
                        stage[ii, pl.ds(16 * jj, 16)] = jnp.maximum(z, 0.0)
                pltpu.sync_copy(stage, agg_sh.at[dxq[q].at[j]], add=True)
                if j == SCH - 1:
                    pltpu.async_copy(
                        dst2_h.at[pl.ds(drow + (sc + 2) * SCH, SCH)],
                        dxq[q], lsem[q])
                    pltpu.async_copy(
                        e_h.at[pl.ds(4 * (ebase + (sc + 2) * SCH * C),
                                     4 * SCH * C)],
                        ebq[q].at[pl.ds(0, 4 * SCH * C)], lsem[q])
        return carry

    lax.fori_loop(0, NSC // 2, pair, 0)

    wait_gather(0)
    wait_sidx(1)
    wait_super(1)

    plsc.subcore_barrier()
    for t in range(RPA // C):
        pltpu.sync_copy(agg_sh.at[pl.ds(r0 + t * C, C)], stage)
        pltpu.sync_copy(stage, out_h.at[pl.ds(c * NP + r0 + t * C, C)])
    rem = RPA - (RPA // C) * C
    pltpu.sync_copy(agg_sh.at[pl.ds(r0 + (RPA // C) * C, rem)],
                    stage.at[pl.ds(0, rem)])
    pltpu.sync_copy(stage.at[pl.ds(0, rem)],
                    out_h.at[pl.ds(c * NP + r0 + (RPA // C) * C, rem)])


def _ka_body(x_ref, s0_ref, s1_ref, cxx_ref, mxe_ref):
    @pl.when(pl.program_id(0) == 0)
    def _():
        cxx_ref[...] = jnp.zeros_like(cxx_ref)
        mxe_ref[...] = jnp.zeros_like(mxe_ref)

    sblk = s0_ref[...] + s1_ref[...]
    xb = x_ref[...]
    w = sblk[:, 4:5]
    cxx_ref[...] += lax.dot_general(xb, xb * w, (((0,), (0,)), ((), ())),
                                    precision=HI,
                                    preferred_element_type=jnp.float32)
    mxe_ref[...] += lax.dot_general(xb, sblk, (((0,), (0,)), ((), ())),
                                    precision=HI,
                                    preferred_element_type=jnp.float32)


def _kb_body(e16_ref, g_ref):
    @pl.when(pl.program_id(0) == 0)
    def _():
        g_ref[...] = jnp.zeros_like(g_ref)

    eb = e16_ref[...]
    g_ref[...] += lax.dot_general(eb, eb, (((0,), (0,)), ((), ())),
                                  precision=HI,
                                  preferred_element_type=jnp.float32)


def _kf_body(cxx_ref, mxe_ref, g_ref, w1_ref, gam_ref, bet_ref,
             w1sp_ref, cpp_ref):
    cxe = mxe_ref[:, 0:4]
    sum_x = mxe_ref[:, 4:5]
    sum_e = g_ref[0:4, 4:5]
    w1 = w1_ref[...]
    w1x = w1[0:128, :]
    w1e = w1[128:132, :]
    dn = (((0,), (0,)), ((), ()))
    t = (lax.dot_general(sum_x, w1x, dn, precision=HI)
         + lax.dot_general(sum_e, w1e, dn, precision=HI))
    top = (lax.dot_general(cxx_ref[...], w1x, (((1,), (0,)), ((), ())),
                           precision=HI)
           + lax.dot_general(cxe, w1e, (((1,), (0,)), ((), ())),
                             precision=HI))
    bot = (lax.dot_general(cxe, w1x, dn, precision=HI)
           + lax.dot_general(g_ref[0:4, 0:4], w1e,
                             (((1,), (0,)), ((), ())), precision=HI))
    ss = (jnp.sum(w1x * top, axis=0, keepdims=True)
          + jnp.sum(w1e * bot, axis=0, keepdims=True))
    tm = t * (1.0 / E)
    var = ss * (1.0 / E) - tm * tm
    sfac = gam_ref[...] * lax.rsqrt(var + 1e-5)
    cp = bet_ref[...] - tm * sfac
    w1s = w1 * sfac
    pad_c = jnp.zeros((132, EMBP - EMB), jnp.float32)
    w1sp_ref[...] = jnp.concatenate([w1s, pad_c], axis=1)
    cp_row = jnp.concatenate([cp, jnp.zeros((1, EMBP - EMB), jnp.float32)],
                             axis=1)
    cpp_ref[...] = jnp.concatenate([cp_row, jnp.zeros((7, EMBP), jnp.float32)],
                                   axis=0)


def _ku_body(x_ref, w1sp_ref, cpp_ref, u_ref):
    u_ref[...] = lax.dot_general(x_ref[...], w1sp_ref[0:128, :],
                                 (((1,), (0,)), ((), ())), precision=HI,
                                 preferred_element_type=jnp.float32) \
                 + cpp_ref[0:1, :]


def _ko_body(a0_ref, a1_ref, d0_ref, d1_ref, x_ref, w2p_ref, b2_ref, h_ref):
    aggb = a0_ref[...] + a1_ref[...]
    deg = d0_ref[:, 4:5] + d1_ref[:, 4:5]
    h = lax.dot_general(aggb, w2p_ref[...], (((1,), (0,)), ((), ())),
                        precision=HI, preferred_element_type=jnp.float32)
    h = h + deg * b2_ref[...]
    h_ref[...] = jnp.where(deg > 0.0, h, x_ref[...])


def kernel(x, edge_index, e, W1, b1, gamma, beta, W2, b2):
    del b1
    src = edge_index[0]
    dst = edge_index[1]
    pad = EPP - E
    sent = jnp.full((pad,), N, jnp.int32)
    src_p = jnp.concatenate([src, sent])
    dst_p = jnp.concatenate([dst, sent])
    e_p = jnp.concatenate(
        [e, jnp.zeros((pad, 4), jnp.float32)]).reshape(-1)
    e16 = jnp.concatenate(
        [e, jnp.ones((E, 1), jnp.float32), jnp.zeros((E, 11), jnp.float32)],
        axis=1)
    e16 = jnp.concatenate([e16, jnp.zeros((EP - E, 16), jnp.float32)])
    x_p = jnp.concatenate([x, jnp.zeros((NP - N, D), jnp.float32)])
    src2 = src_p.reshape(-1, C1)
    dst2c1 = dst_p.reshape(-1, C1)
    dst2 = dst_p.reshape(-1, C)

    sd = _sc_stats(src2, dst2c1, e16)

    cxx, mxe = pl.pallas_call(
        _ka_body,
        grid=(NBLK,),
        in_specs=[
            pl.BlockSpec((BLK, D), lambda i: (i, 0)),
            pl.BlockSpec((BLK, 16), lambda i: (i, 0)),
            pl.BlockSpec((BLK, 16), lambda i: (i + NBLK, 0)),
        ],
        out_specs=[
            pl.BlockSpec((D, D), lambda i: (0, 0)),
            pl.BlockSpec((D, 16), lambda i: (0, 0)),
        ],
        out_shape=[
            jax.ShapeDtypeStruct((D, D), jnp.float32),
            jax.ShapeDtypeStruct((D, 16), jnp.float32),
        ],
    )(x_p, sd, sd)

    g = pl.pallas_call(
        _kb_body,
        grid=(EP // EBLK,),
        in_specs=[pl.BlockSpec((EBLK, 16), lambda i: (i, 0))],
        out_specs=pl.BlockSpec((16, 16), lambda i: (0, 0)),
        out_shape=jax.ShapeDtypeStruct((16, 16), jnp.float32),
    )(e16)

    w1sp, cpp = pl.pallas_call(
        _kf_body,
        out_shape=[
            jax.ShapeDtypeStruct((EMB, EMBP), jnp.float32),
            jax.ShapeDtypeStruct((8, EMBP), jnp.float32),
        ],
    )(cxx, mxe, g, W1, gamma.reshape(1, EMB), beta.reshape(1, EMB))

    u = pl.pallas_call(
        _ku_body,
        grid=(NBLK,),
        in_specs=[
            pl.BlockSpec((BLK, D), lambda i: (i, 0)),
            pl.BlockSpec((EMB, EMBP), lambda i: (0, 0)),
            pl.BlockSpec((8, EMBP), lambda i: (0, 0)),
        ],
        out_specs=pl.BlockSpec((BLK, EMBP), lambda i: (i, 0)),
        out_shape=jax.ShapeDtypeStruct((NP, EMBP), jnp.float32),
    )(x_p, w1sp, cpp)

    w1e_s = w1sp[128:132, :]
    agg = _sc_edges(u, src_p, dst2, e_p, w1e_s)

    w2p = jnp.concatenate([W2, jnp.zeros((EMBP - EMB, D), jnp.float32)])
    h = pl.pallas_call(
        _ko_body,
        grid=(NBLK,),
        in_specs=[
            pl.BlockSpec((BLK, EMBP), lambda i: (i, 0)),
            pl.BlockSpec((BLK, EMBP), lambda i: (i + NBLK, 0)),
            pl.BlockSpec((BLK, 16), lambda i: (i + 2 * NBLK, 0)),
            pl.BlockSpec((BLK, 16), lambda i: (i + 3 * NBLK, 0)),
            pl.BlockSpec((BLK, D), lambda i: (i, 0)),
            pl.BlockSpec((EMBP, D), lambda i: (0, 0)),
            pl.BlockSpec((1, D), lambda i: (0, 0)),
        ],
        out_specs=pl.BlockSpec((BLK, D), lambda i: (i, 0)),
        out_shape=jax.ShapeDtypeStruct((NP, D), jnp.float32),
    )(agg, agg, sd, sd, x_p, w2p, b2.reshape(1, D))

    return h[:N]
